# baseline (speedup 1.0000x reference)
import functools

import jax
import jax.numpy as jnp
from jax import lax
from jax.experimental import pallas as pl
from jax.experimental.pallas import tpu as pltpu
from jax.experimental.pallas import tpu_sc as plsc

_DIM = 512
_HID = 512
_E = 16
_LIMIT = 7.0
_TOK = 64


def _gate_body(x_ref, gwt_ref, gb_ref, g_ref):
    dnt = (((1,), (1,)), ((), ()))
    g_ref[...] = (
        lax.dot_general(x_ref[...], gwt_ref[...], dnt,
                        preferred_element_type=jnp.float32)
        + gb_ref[...]
    )


def _gate_call(x, gate_w, gate_b):
    return pl.pallas_call(
        _gate_body,
        out_shape=jax.ShapeDtypeStruct((_TOK, _E), jnp.float32),
    )(x, gate_w.T, gate_b)


def _route_body(g_hbm, p_hbm, g_v, p_v):
    wid = lax.axis_index("s")
    base = wid * 4
    pltpu.sync_copy(g_hbm.at[pl.ds(base, 4)], g_v)
    iota = lax.iota(jnp.int32, 16)
    neg = jnp.float32(jnp.finfo(jnp.float32).min)

    gdn = lax.GatherDimensionNumbers(
        offset_dims=(), collapsed_slice_dims=(0,), start_index_map=(0,)
    )

    def _perm(u, idx):
        return lax.gather(
            u, idx[:, None], gdn, slice_sizes=(1,),
            mode=lax.GatherScatterMode.PROMISE_IN_BOUNDS,
        )

    def _bfly(u, op):
        for s in (8, 4, 2, 1):
            u = op(u, _perm(u, iota ^ s))
        return u

    for j in range(4):
        v = g_v[j]
        m1 = _bfly(v, jnp.maximum)
        i1 = _bfly(jnp.where(v == m1, iota, _E), jnp.minimum)
        mask1 = iota == i1
        v2 = jnp.where(mask1, neg, v)
        m2 = _bfly(v2, jnp.maximum)
        i2 = _bfly(jnp.where(v2 == m2, iota, _E), jnp.minimum)
        mask2 = iota == i2
        a = jnp.exp(m2 - m1)
        p1 = 1.0 / (1.0 + a)
        p2 = a / (1.0 + a)
        row = jnp.where(mask1, p1, jnp.where(mask2, p2, jnp.float32(0.0)))
        p_v[j] = row
    pltpu.sync_copy(p_v, p_hbm.at[pl.ds(base, 4)])


def _route_call(g):
    mesh = plsc.VectorSubcoreMesh(
        core_axis_name="c", subcore_axis_name="s", num_cores=1
    )
    kern = pl.kernel(
        _route_body,
        mesh=mesh,
        out_type=jax.ShapeDtypeStruct((_TOK, _E), jnp.float32),
        scratch_types=[
            pltpu.VMEM((4, _E), jnp.float32),
            pltpu.VMEM((4, _E), jnp.float32),
        ],
    )
    return kern(g)


_EPB = 8


def _ffn_body(x_ref, p_ref, w1_ref, b1_ref, w2_ref, b2_ref, out_ref,
              sg_ref, sl_ref):
    eb = pl.program_id(0)

    @pl.when(eb == 0)
    def _():
        ri = lax.broadcasted_iota(jnp.int32, (2 * _HID, _HID), 0)
        ci = lax.broadcasted_iota(jnp.int32, (2 * _HID, _HID), 1)
        sg_ref[...] = (ri == 2 * ci).astype(jnp.bfloat16)
        sl_ref[...] = (ri == 2 * ci + 1).astype(jnp.bfloat16)

    x = x_ref[...].astype(jnp.bfloat16)
    dnt = (((1,), (1,)), ((), ()))
    dnk = (((1,), (0,)), ((), ()))
    lane = lax.broadcasted_iota(jnp.int32, (_TOK, _E), 1)
    acc = None
    for es in range(_EPB):
        e = eb * _EPB + es
        w1e = w1_ref[es].astype(jnp.bfloat16)
        h = lax.dot_general(x, w1e, dnt, preferred_element_type=jnp.float32)
        h = (h + b1_ref[pl.ds(e, 1), :]).astype(jnp.bfloat16)
        hg = lax.dot_general(h, sg_ref[...], dnk,
                             preferred_element_type=jnp.float32)
        hl = lax.dot_general(h, sl_ref[...], dnk,
                             preferred_element_type=jnp.float32)
        hg = jnp.minimum(hg, _LIMIT)
        hl = jnp.clip(hl, -_LIMIT, _LIMIT)
        act = (hg * jax.nn.sigmoid(1.702 * hg) * (hl + 1.0)).astype(jnp.bfloat16)
        w2e = w2_ref[es].astype(jnp.bfloat16)
        y = lax.dot_general(act, w2e, dnt, preferred_element_type=jnp.float32)
        y = y + b2_ref[pl.ds(e, 1), :]
        p = jnp.sum(jnp.where(lane == e, p_ref[...], 0.0), axis=1,
                    keepdims=True)
        acc = p * y if acc is None else acc + p * y

    @pl.when(eb == 0)
    def _():
        out_ref[...] = acc

    @pl.when(eb > 0)
    def _():
        out_ref[...] += acc


def _ffn_call(x, p, w1, b1, w2, b2):
    return pl.pallas_call(
        _ffn_body,
        grid=(_E // _EPB,),
        in_specs=[
            pl.BlockSpec((_TOK, _DIM), lambda e: (0, 0)),
            pl.BlockSpec((_TOK, _E), lambda e: (0, 0)),
            pl.BlockSpec((_EPB, 2 * _HID, _DIM), lambda e: (e, 0, 0)),
            pl.BlockSpec((_E, 2 * _HID), lambda e: (0, 0)),
            pl.BlockSpec((_EPB, _DIM, _HID), lambda e: (e, 0, 0)),
            pl.BlockSpec((_E, _DIM), lambda e: (0, 0)),
        ],
        out_specs=pl.BlockSpec((_TOK, _DIM), lambda e: (0, 0)),
        out_shape=jax.ShapeDtypeStruct((_TOK, _DIM), jnp.float32),
        scratch_shapes=[
            pltpu.VMEM((2 * _HID, _HID), jnp.bfloat16),
            pltpu.VMEM((2 * _HID, _HID), jnp.bfloat16),
        ],
        compiler_params=pltpu.CompilerParams(
            dimension_semantics=("arbitrary",),
        ),
    )(x, p, w1, b1, w2, b2)


def kernel(x, gate_w, gate_b, w1, b1, w2, b2):
    g = _gate_call(x, gate_w, gate_b)
    p = _route_call(g)
    return _ffn_call(x, p, w1, b1, w2, b2)


# speedup vs baseline: 1.0530x; 1.0530x over previous
import functools

import jax
import jax.numpy as jnp
from jax import lax
from jax.experimental import pallas as pl
from jax.experimental.pallas import tpu as pltpu
from jax.experimental.pallas import tpu_sc as plsc

_DIM = 512
_HID = 512
_E = 16
_LIMIT = 7.0
_TOK = 64


def _gate_body(x_ref, gwt_ref, gb_ref, g_ref):
    dnt = (((1,), (1,)), ((), ()))
    g_ref[...] = (
        lax.dot_general(x_ref[...], gwt_ref[...], dnt,
                        preferred_element_type=jnp.float32)
        + gb_ref[...]
    )


def _gate_call(x, gate_w, gate_b):
    return pl.pallas_call(
        _gate_body,
        out_shape=jax.ShapeDtypeStruct((_TOK, _E), jnp.float32),
    )(x, gate_w.T, gate_b)


def _route_body(g_hbm, p_hbm, g_v, p_v):
    wid = lax.axis_index("s")
    base = wid * 4
    pltpu.sync_copy(g_hbm.at[pl.ds(base, 4)], g_v)
    iota = lax.iota(jnp.int32, 16)
    neg = jnp.float32(jnp.finfo(jnp.float32).min)

    gdn = lax.GatherDimensionNumbers(
        offset_dims=(), collapsed_slice_dims=(0,), start_index_map=(0,)
    )

    def _perm(u, idx):
        return lax.gather(
            u, idx[:, None], gdn, slice_sizes=(1,),
            mode=lax.GatherScatterMode.PROMISE_IN_BOUNDS,
        )

    def _bfly(u, op):
        for s in (8, 4, 2, 1):
            u = op(u, _perm(u, iota ^ s))
        return u

    for j in range(4):
        v = g_v[j]
        m1 = _bfly(v, jnp.maximum)
        i1 = _bfly(jnp.where(v == m1, iota, _E), jnp.minimum)
        mask1 = iota == i1
        v2 = jnp.where(mask1, neg, v)
        m2 = _bfly(v2, jnp.maximum)
        i2 = _bfly(jnp.where(v2 == m2, iota, _E), jnp.minimum)
        mask2 = iota == i2
        a = jnp.exp(m2 - m1)
        p1 = 1.0 / (1.0 + a)
        p2 = a / (1.0 + a)
        row = jnp.where(mask1, p1, jnp.where(mask2, p2, jnp.float32(0.0)))
        p_v[j] = row
    pltpu.sync_copy(p_v, p_hbm.at[pl.ds(base, 4)])


def _route_call(g):
    mesh = plsc.VectorSubcoreMesh(
        core_axis_name="c", subcore_axis_name="s", num_cores=1
    )
    kern = pl.kernel(
        _route_body,
        mesh=mesh,
        out_type=jax.ShapeDtypeStruct((_TOK, _E), jnp.float32),
        scratch_types=[
            pltpu.VMEM((4, _E), jnp.float32),
            pltpu.VMEM((4, _E), jnp.float32),
        ],
    )
    return kern(g)


_EPB = 4


def _ffn_body(x_ref, p_ref, w1_ref, b1_ref, w2_ref, b2_ref, out_ref,
              sg_ref, sl_ref):
    eb = pl.program_id(0)

    @pl.when(eb == 0)
    def _():
        ri = lax.broadcasted_iota(jnp.int32, (2 * _HID, _HID), 0)
        ci = lax.broadcasted_iota(jnp.int32, (2 * _HID, _HID), 1)
        sg_ref[...] = (ri == 2 * ci).astype(jnp.bfloat16)
        sl_ref[...] = (ri == 2 * ci + 1).astype(jnp.bfloat16)

    x = x_ref[...].astype(jnp.bfloat16)
    dnt = (((1,), (1,)), ((), ()))
    dnk = (((1,), (0,)), ((), ()))
    lane = lax.broadcasted_iota(jnp.int32, (_TOK, _E), 1)
    acc = None
    for es in range(_EPB):
        e = eb * _EPB + es
        w1e = w1_ref[es].astype(jnp.bfloat16)
        h = lax.dot_general(x, w1e, dnt, preferred_element_type=jnp.float32)
        h = (h + b1_ref[pl.ds(e, 1), :]).astype(jnp.bfloat16)
        hg = lax.dot_general(h, sg_ref[...], dnk,
                             preferred_element_type=jnp.float32)
        hl = lax.dot_general(h, sl_ref[...], dnk,
                             preferred_element_type=jnp.float32)
        hg = jnp.minimum(hg, _LIMIT)
        hl = jnp.clip(hl, -_LIMIT, _LIMIT)
        act = (hg * jax.nn.sigmoid(1.702 * hg) * (hl + 1.0)).astype(jnp.bfloat16)
        w2e = w2_ref[es].astype(jnp.bfloat16)
        y = lax.dot_general(act, w2e, dnt, preferred_element_type=jnp.float32)
        y = y + b2_ref[pl.ds(e, 1), :]
        p = jnp.sum(jnp.where(lane == e, p_ref[...], 0.0), axis=1,
                    keepdims=True)
        acc = p * y if acc is None else acc + p * y

    @pl.when(eb == 0)
    def _():
        out_ref[...] = acc

    @pl.when(eb > 0)
    def _():
        out_ref[...] += acc


def _ffn_call(x, p, w1, b1, w2, b2):
    return pl.pallas_call(
        _ffn_body,
        grid=(_E // _EPB,),
        in_specs=[
            pl.BlockSpec((_TOK, _DIM), lambda e: (0, 0)),
            pl.BlockSpec((_TOK, _E), lambda e: (0, 0)),
            pl.BlockSpec((_EPB, 2 * _HID, _DIM), lambda e: (e, 0, 0)),
            pl.BlockSpec((_E, 2 * _HID), lambda e: (0, 0)),
            pl.BlockSpec((_EPB, _DIM, _HID), lambda e: (e, 0, 0)),
            pl.BlockSpec((_E, _DIM), lambda e: (0, 0)),
        ],
        out_specs=pl.BlockSpec((_TOK, _DIM), lambda e: (0, 0)),
        out_shape=jax.ShapeDtypeStruct((_TOK, _DIM), jnp.float32),
        scratch_shapes=[
            pltpu.VMEM((2 * _HID, _HID), jnp.bfloat16),
            pltpu.VMEM((2 * _HID, _HID), jnp.bfloat16),
        ],
        compiler_params=pltpu.CompilerParams(
            dimension_semantics=("arbitrary",),
        ),
    )(x, p, w1, b1, w2, b2)


def kernel(x, gate_w, gate_b, w1, b1, w2, b2):
    g = _gate_call(x, gate_w, gate_b)
    p = _route_call(g)
    return _ffn_call(x, p, w1, b1, w2, b2)


# speedup vs baseline: 1.0568x; 1.0036x over previous
import jax
import jax.numpy as jnp
from jax import lax
from jax.experimental import pallas as pl
from jax.experimental.pallas import tpu as pltpu
from jax.experimental.pallas import tpu_sc as plsc

_DIM = 512
_HID = 512
_E = 16
_LIMIT = 7.0
_TOK = 64


def _gate_body(x_ref, gwt_ref, gb_ref, g_ref):
    dnt = (((1,), (1,)), ((), ()))
    g_ref[...] = (
        lax.dot_general(x_ref[...], gwt_ref[...], dnt,
                        preferred_element_type=jnp.float32)
        + gb_ref[...]
    )


def _gate_call(x, gate_w, gate_b):
    return pl.pallas_call(
        _gate_body,
        out_shape=jax.ShapeDtypeStruct((_TOK, _E), jnp.float32),
    )(x, gate_w.T, gate_b)


def _route_body(g_hbm, p_hbm, g_v, p_v):
    wid = lax.axis_index("s")
    base = wid * 4
    pltpu.sync_copy(g_hbm.at[pl.ds(base, 4)], g_v)
    iota = lax.iota(jnp.int32, 16)
    neg = jnp.float32(jnp.finfo(jnp.float32).min)

    gdn = lax.GatherDimensionNumbers(
        offset_dims=(), collapsed_slice_dims=(0,), start_index_map=(0,)
    )

    def _perm(u, idx):
        return lax.gather(
            u, idx[:, None], gdn, slice_sizes=(1,),
            mode=lax.GatherScatterMode.PROMISE_IN_BOUNDS,
        )

    def _bfly(u, op):
        for s in (8, 4, 2, 1):
            u = op(u, _perm(u, iota ^ s))
        return u

    for j in range(4):
        v = g_v[j]
        m1 = _bfly(v, jnp.maximum)
        i1 = _bfly(jnp.where(v == m1, iota, _E), jnp.minimum)
        mask1 = iota == i1
        v2 = jnp.where(mask1, neg, v)
        m2 = _bfly(v2, jnp.maximum)
        i2 = _bfly(jnp.where(v2 == m2, iota, _E), jnp.minimum)
        mask2 = iota == i2
        a = jnp.exp(m2 - m1)
        p1 = 1.0 / (1.0 + a)
        p2 = a / (1.0 + a)
        row = jnp.where(mask1, p1, jnp.where(mask2, p2, jnp.float32(0.0)))
        p_v[j] = row
    pltpu.sync_copy(p_v, p_hbm.at[pl.ds(base, 4)])


def _route_call(g):
    mesh = plsc.VectorSubcoreMesh(
        core_axis_name="c", subcore_axis_name="s", num_cores=1
    )
    kern = pl.kernel(
        _route_body,
        mesh=mesh,
        out_type=jax.ShapeDtypeStruct((_TOK, _E), jnp.float32),
        scratch_types=[
            pltpu.VMEM((4, _E), jnp.float32),
            pltpu.VMEM((4, _E), jnp.float32),
        ],
    )
    return kern(g)


_EPB = 4


def _ffn_body(x_ref, p_ref, w1_ref, b1_ref, w2_ref, b2_ref, out_ref,
              sg_ref, sl_ref):
    eb = pl.program_id(0)

    @pl.when(eb == 0)
    def _():
        ri = lax.broadcasted_iota(jnp.int32, (2 * _HID, _HID), 0)
        ci = lax.broadcasted_iota(jnp.int32, (2 * _HID, _HID), 1)
        sg_ref[...] = (ri == 2 * ci).astype(jnp.bfloat16)
        sl_ref[...] = (ri == 2 * ci + 1).astype(jnp.bfloat16)

    x = x_ref[...].astype(jnp.bfloat16)
    dnt = (((1,), (1,)), ((), ()))
    dnk = (((1,), (0,)), ((), ()))
    lane = lax.broadcasted_iota(jnp.int32, (_TOK, _E), 1)
    acc = None
    for es in range(_EPB):
        e = eb * _EPB + es
        w1e = w1_ref[es].astype(jnp.bfloat16)
        h = lax.dot_general(x, w1e, dnt, preferred_element_type=jnp.float32)
        h = (h + b1_ref[pl.ds(e, 1), :]).astype(jnp.bfloat16)
        hg = lax.dot_general(h, sg_ref[...], dnk,
                             preferred_element_type=jnp.float32)
        hl = lax.dot_general(h, sl_ref[...], dnk,
                             preferred_element_type=jnp.float32)
        hg = jnp.minimum(hg, _LIMIT)
        hl = jnp.clip(hl, -_LIMIT, _LIMIT)
        act = (hg * jax.nn.sigmoid(1.702 * hg) * (hl + 1.0)).astype(jnp.bfloat16)
        w2e = w2_ref[es].astype(jnp.bfloat16)
        y = lax.dot_general(act, w2e, dnt, preferred_element_type=jnp.float32)
        y = y + b2_ref[pl.ds(e, 1), :]
        p = jnp.sum(jnp.where(lane == e, p_ref[...], 0.0), axis=1,
                    keepdims=True)
        acc = p * y if acc is None else acc + p * y

    @pl.when(eb == 0)
    def _():
        out_ref[...] = acc

    @pl.when(eb > 0)
    def _():
        out_ref[...] += acc


def _ffn_call(x, p, w1, b1, w2, b2):
    return pl.pallas_call(
        _ffn_body,
        grid=(_E // _EPB,),
        in_specs=[
            pl.BlockSpec((_TOK, _DIM), lambda e: (0, 0)),
            pl.BlockSpec((_TOK, _E), lambda e: (0, 0)),
            pl.BlockSpec((_EPB, 2 * _HID, _DIM), lambda e: (e, 0, 0)),
            pl.BlockSpec((_E, 2 * _HID), lambda e: (0, 0)),
            pl.BlockSpec((_EPB, _DIM, _HID), lambda e: (e, 0, 0)),
            pl.BlockSpec((_E, _DIM), lambda e: (0, 0)),
        ],
        out_specs=pl.BlockSpec((_TOK, _DIM), lambda e: (0, 0)),
        out_shape=jax.ShapeDtypeStruct((_TOK, _DIM), jnp.float32),
        scratch_shapes=[
            pltpu.VMEM((2 * _HID, _HID), jnp.bfloat16),
            pltpu.VMEM((2 * _HID, _HID), jnp.bfloat16),
        ],
        compiler_params=pltpu.CompilerParams(
            dimension_semantics=("arbitrary",),
        ),
    )(x, p, w1, b1, w2, b2)


def kernel(x, gate_w, gate_b, w1, b1, w2, b2):
    g = _gate_call(x, gate_w, gate_b)
    p = _route_call(g)
    return _ffn_call(x, p, w1, b1, w2, b2)
